# baseline (speedup 1.0000x reference)
import functools
from itertools import combinations

import numpy as np
import jax
import jax.numpy as jnp
from jax import lax
from jax.experimental import pallas as pl
from jax.experimental.pallas import tpu as pltpu
from jax.experimental.pallas import tpu_sc as plsc

_TF = 128
_PBLK = 128

_NC = 256
_NPAIR = 32640
_NCOPY = 8
_NQ = 4
_KQ = _NPAIR // _NQ
_NT_STAGE = 65
_P = 32897


def _multihot_rows(num_classes, max_set_size):
    rows = []
    for k in range(max_set_size + 1):
        for comb in combinations(range(num_classes), k):
            row = np.zeros(num_classes, dtype=np.float32)
            if comb:
                row[list(comb)] = 1.0
            rows.append(row)
    return np.stack(rows, axis=0)


@functools.lru_cache(maxsize=1)
def _sc_tables():
    a_ar, b_ar = np.triu_indices(_NC, k=1)
    assert a_ar.size == _NPAIR
    copy = np.zeros(_NPAIR, np.int64)
    for k0 in range(0, _NPAIR, 16):
        seen = {}
        for i, bb in enumerate(b_ar[k0:k0 + 16]):
            c = seen.get(bb, 0)
            copy[k0 + i] = c
            seen[bb] = c + 1
    assert copy.max() < _NCOPY
    cidx = (copy * _NC + b_ar).astype(np.int32)
    a_range = np.arange(_NC, dtype=np.int64)
    off = a_range * 255 - a_range * (a_range - 1) // 2
    end_g = off + (255 - a_range) - 1
    start_g = off
    endsq = np.zeros((_NQ, _NC), np.int32)
    startsq = np.zeros((_NQ, _NC), np.int32)
    for q in range(_NQ):
        k0, k1 = q * _KQ, (q + 1) * _KQ
        endsq[q] = (np.clip(end_g, k0 - 1, k1 - 1) - k0).astype(np.int32)
        startsq[q] = (np.clip(start_g - 1, k0 - 1, k1 - 1) - k0).astype(np.int32)
    return cidx, endsq.reshape(-1), startsq.reshape(-1)


def _tc_kernel(x2, T, P, C):
    PM = ((P - 1) // _PBLK) * _PBLK
    W = P - PM
    assert W == 1
    mnp = _multihot_rows(C, 2)
    assert mnp.shape == (P, C)
    m_bf16 = jnp.asarray(mnp[:PM], dtype=jnp.bfloat16)
    mt = jnp.asarray(mnp[PM:])

    def body(x_ref, m_ref, mt_ref, o_ref):
        x = x_ref[...]
        e = jnp.exp(x[:, :PM]).astype(jnp.bfloat16)
        acc = jax.lax.dot_general(
            e, m_ref[...], (((1,), (0,)), ((), ())),
            preferred_element_type=jnp.float32)
        et = jnp.exp(x[:, PM:])
        o_ref[...] = acc + et * mt_ref[...]

    return pl.pallas_call(
        body,
        grid=(T // _TF,),
        in_specs=[
            pl.BlockSpec((_TF, P), lambda f: (f, 0)),
            pl.BlockSpec((PM, C), lambda f: (0, 0)),
            pl.BlockSpec((W, C), lambda f: (0, 0)),
        ],
        out_specs=pl.BlockSpec((_TF, C), lambda f: (f, 0)),
        out_shape=jax.ShapeDtypeStruct((T, C), jnp.float32),
    )(x2, m_bf16, mt)


def _sc_kernel(x3, T, C):
    cidx, endsq, startsq = _sc_tables()
    cidx_in = jnp.asarray(cidx)
    ends_in = jnp.asarray(endsq)
    starts_in = jnp.asarray(startsq)

    info = plsc.get_sparse_core_info()
    nw = info.num_cores * info.num_subcores
    nslab = T // 8
    ntask = nslab * _NQ
    assert ntask % nw == 0
    per = ntask // nw
    niter = _KQ // 32

    mesh = plsc.VectorSubcoreMesh(core_axis_name="c", subcore_axis_name="s")

    @functools.partial(
        pl.kernel, mesh=mesh,
        compiler_params=pltpu.CompilerParams(needs_layout_passes=False),
        out_type=jax.ShapeDtypeStruct((_NQ, T, C), jnp.float32),
        scratch_types=[
            pltpu.VMEM((_NT_STAGE, 8, 128), jnp.float32),
            pltpu.VMEM((3, 8, 128), jnp.float32),
            pltpu.VMEM((_KQ + 32,), jnp.float32),
            pltpu.VMEM((_KQ + 32,), jnp.int32),
            pltpu.VMEM((_NCOPY * _NC,), jnp.float32),
            pltpu.VMEM((_NC,), jnp.int32),
            pltpu.VMEM((_NC,), jnp.int32),
            pltpu.VMEM((2, 8, 128), jnp.float32),
            pltpu.SemaphoreType.DMA,
        ],
    )
    def k(x_hbm, cidx_hbm, ends_hbm, starts_hbm, out_hbm,
          xq, sb, pbuf, cidxv, accf, endsv, startsv, orow, sem):
        wid = lax.axis_index("s") * info.num_cores + lax.axis_index("c")
        qd = wid & 3
        k0 = qd * _KQ
        cstart = ((257 + k0) >> 7) << 7
        base0 = 257 + k0 - cstart
        pmax = jnp.where(qd == _NQ - 1, _KQ * _NQ - 1 + 257 - cstart,
                         _NT_STAGE * 128)
        pltpu.sync_copy(cidx_hbm.at[pl.ds(k0, _KQ)], cidxv.at[pl.ds(0, _KQ)])
        pltpu.sync_copy(ends_hbm.at[pl.ds(qd * _NC, _NC)], endsv)
        pltpu.sync_copy(starts_hbm.at[pl.ds(qd * _NC, _NC)], startsv)
        iota = lax.iota(jnp.int32, 16)
        zero16 = jnp.zeros((16,), jnp.float32)
        pmax_v = jnp.full((16,), 0, jnp.int32) + pmax

        def task_body(i, _):
            g = i * 8 + (wid >> 2)
            copies = []
            for j in range(_NT_STAGE):
                csrc = pl.multiple_of(
                    jnp.minimum(cstart + j * 128, _P - 1 - 128), 128)
                copies.append(pltpu.async_copy(
                    x_hbm.at[g, :, pl.ds(csrc, 128)], xq.at[j], sem))
            for j in range(3):
                copies.append(pltpu.async_copy(
                    x_hbm.at[g, :, pl.ds(j * 128, 128)], sb.at[j], sem))
            for c in copies:
                c.wait()

            def frame_body(f, _):
                for q in range(_NCOPY * _NC // 16):
                    accf[pl.ds(q * 16, 16)] = zero16
                fv = jnp.full((16,), 0, jnp.int32) + f

                def chunk_body(kk, carry):
                    kloc = kk * 32
                    res = carry
                    for h in range(2):
                        lo = cidxv[pl.ds(kloc + h * 16, 16)]
                        posv = base0 + kloc + h * 16 + iota
                        m = posv < pmax_v
                        jv = lax.shift_right_logical(posv, 7)
                        lv = posv & 127
                        gat = plsc.load_gather(xq, [jv, fv, lv], mask=m)
                        e = jnp.where(m, jnp.exp(gat), 0.0)
                        plsc.addupdate_scatter(accf, [lo], e)
                        pbuf[pl.ds(kloc + h * 16, 16)] = plsc.cumsum(e) + res
                        res = res + jnp.sum(e)
                    return res

                lax.fori_loop(0, niter, chunk_body, jnp.float32(0.0),
                              unroll=2)

                for q in range(C // 16):
                    ei = endsv[pl.ds(q * 16, 16)]
                    si = startsv[pl.ds(q * 16, 16)]
                    me = ei >= 0
                    ms = si >= 0
                    ge = jnp.where(me, plsc.load_gather(pbuf, [ei], mask=me),
                                   0.0)
                    gs = jnp.where(ms, plsc.load_gather(pbuf, [si], mask=ms),
                                   0.0)
                    col = accf[pl.ds(q * 16, 16)]
                    for cp in range(1, _NCOPY):
                        col = col + accf[pl.ds(cp * _NC + q * 16, 16)]
                    jj, l0 = divmod(q * 16, 128)
                    orow[jj, f, pl.ds(l0, 16)] = (ge - gs) + col

                @pl.when(qd == 0)
                def _():
                    for q in range(C // 16):
                        spos = 1 + q * 16 + iota
                        sjv = lax.shift_right_logical(spos, 7)
                        slv = spos & 127
                        se = jnp.exp(plsc.load_gather(sb, [sjv, fv, slv]))
                        jj, l0 = divmod(q * 16, 128)
                        orow[jj, f, pl.ds(l0, 16)] += se
                return 0

            lax.fori_loop(0, 8, frame_body, 0)
            pltpu.sync_copy(orow.at[0],
                            out_hbm.at[qd, pl.ds(g * 8, 8), pl.ds(0, 128)])
            pltpu.sync_copy(orow.at[1],
                            out_hbm.at[qd, pl.ds(g * 8, 8), pl.ds(128, 128)])
            return 0

        lax.fori_loop(0, per, task_body, 0)

    return k(x3, cidx_in, ends_in, starts_in)


def kernel(powerset, mapping):
    B, T, P = powerset.shape
    _, C = mapping.shape
    assert P == _P and C == _NC
    x3 = powerset.reshape(T // 8, 8, P)
    parts = _sc_kernel(x3, T, C)
    out = parts[0] + parts[1] + parts[2] + parts[3]
    mnp = _multihot_rows(C, 2)
    mt = jnp.asarray(mnp[P - 1:])
    out = out + jnp.exp(powerset.reshape(T, P)[:, P - 1:]) * mt
    return out.reshape(B, T, C)


# speedup vs baseline: 7.9327x; 7.9327x over previous
import functools
from itertools import combinations

import numpy as np
import jax
import jax.numpy as jnp
from jax import lax
from jax.experimental import pallas as pl
from jax.experimental.pallas import tpu as pltpu
from jax.experimental.pallas import tpu_sc as plsc

_TF = 128
_PBLK = 128

_NC = 256
_NPAIR = 32640
_NCOPY = 8
_NQ = 4
_KQ = _NPAIR // _NQ
_NT_STAGE = 65
_P = 32897


def _multihot_rows(num_classes, max_set_size):
    rows = []
    for k in range(max_set_size + 1):
        for comb in combinations(range(num_classes), k):
            row = np.zeros(num_classes, dtype=np.float32)
            if comb:
                row[list(comb)] = 1.0
            rows.append(row)
    return np.stack(rows, axis=0)


@functools.lru_cache(maxsize=1)
def _sc_tables():
    a_ar, b_ar = np.triu_indices(_NC, k=1)
    assert a_ar.size == _NPAIR
    copy = np.zeros(_NPAIR, np.int64)
    for k0 in range(0, _NPAIR, 16):
        seen = {}
        for i, bb in enumerate(b_ar[k0:k0 + 16]):
            c = seen.get(bb, 0)
            copy[k0 + i] = c
            seen[bb] = c + 1
    assert copy.max() < _NCOPY
    cidx = (copy * _NC + b_ar).astype(np.int32)
    a_range = np.arange(_NC, dtype=np.int64)
    off = a_range * 255 - a_range * (a_range - 1) // 2
    end_g = off + (255 - a_range) - 1
    start_g = off
    endsq = np.zeros((_NQ, _NC), np.int32)
    startsq = np.zeros((_NQ, _NC), np.int32)
    for q in range(_NQ):
        k0, k1 = q * _KQ, (q + 1) * _KQ
        endsq[q] = (np.clip(end_g, k0 - 1, k1 - 1) - k0).astype(np.int32)
        startsq[q] = (np.clip(start_g - 1, k0 - 1, k1 - 1) - k0).astype(np.int32)
    return cidx, endsq.reshape(-1), startsq.reshape(-1)


def _tc_kernel(x2, T, P, C):
    PM = ((P - 1) // _PBLK) * _PBLK
    W = P - PM
    assert W == 1
    mnp = _multihot_rows(C, 2)
    assert mnp.shape == (P, C)
    m_bf16 = jnp.asarray(mnp[:PM], dtype=jnp.bfloat16)
    mt = jnp.asarray(mnp[PM:])

    def body(x_ref, m_ref, mt_ref, o_ref):
        x = x_ref[...]
        e = jnp.exp(x[:, :PM]).astype(jnp.bfloat16)
        acc = jax.lax.dot_general(
            e, m_ref[...], (((1,), (0,)), ((), ())),
            preferred_element_type=jnp.float32)
        et = jnp.exp(x[:, PM:])
        o_ref[...] = acc + et * mt_ref[...]

    return pl.pallas_call(
        body,
        grid=(T // _TF,),
        in_specs=[
            pl.BlockSpec((_TF, P), lambda f: (f, 0)),
            pl.BlockSpec((PM, C), lambda f: (0, 0)),
            pl.BlockSpec((W, C), lambda f: (0, 0)),
        ],
        out_specs=pl.BlockSpec((_TF, C), lambda f: (f, 0)),
        out_shape=jax.ShapeDtypeStruct((T, C), jnp.float32),
    )(x2, m_bf16, mt)


def _sc_kernel(x3, T, C, g0=0):
    cidx, endsq, startsq = _sc_tables()
    cidx_in = jnp.asarray(cidx)
    ends_in = jnp.asarray(endsq)
    starts_in = jnp.asarray(startsq)

    info = plsc.get_sparse_core_info()
    nw = info.num_cores * info.num_subcores
    nslab = T // 8
    ntask = nslab * _NQ
    assert ntask % nw == 0
    per = ntask // nw
    niter = _KQ // 32

    mesh = plsc.VectorSubcoreMesh(core_axis_name="c", subcore_axis_name="s")

    @functools.partial(
        pl.kernel, mesh=mesh,
        compiler_params=pltpu.CompilerParams(needs_layout_passes=False),
        out_type=jax.ShapeDtypeStruct((_NQ, T, C), jnp.float32),
        scratch_types=[
            pltpu.VMEM((_NT_STAGE, 8, 128), jnp.float32),
            pltpu.VMEM((3, 8, 128), jnp.float32),
            pltpu.VMEM((_KQ + 32,), jnp.float32),
            pltpu.VMEM((_KQ + 32,), jnp.int32),
            pltpu.VMEM((_NCOPY * _NC,), jnp.float32),
            pltpu.VMEM((_NC,), jnp.int32),
            pltpu.VMEM((_NC,), jnp.int32),
            pltpu.VMEM((2, 8, 128), jnp.float32),
            pltpu.SemaphoreType.DMA,
        ],
    )
    def k(x_hbm, cidx_hbm, ends_hbm, starts_hbm, out_hbm,
          xq, sb, pbuf, cidxv, accf, endsv, startsv, orow, sem):
        wid = lax.axis_index("s") * info.num_cores + lax.axis_index("c")
        qd = wid & 3
        k0 = qd * _KQ
        cstart = ((257 + k0) >> 7) << 7
        base0 = 257 + k0 - cstart
        pmax = jnp.where(qd == _NQ - 1, _KQ * _NQ - 1 + 257 - cstart,
                         _NT_STAGE * 128)
        pltpu.sync_copy(cidx_hbm.at[pl.ds(k0, _KQ)], cidxv.at[pl.ds(0, _KQ)])
        pltpu.sync_copy(ends_hbm.at[pl.ds(qd * _NC, _NC)], endsv)
        pltpu.sync_copy(starts_hbm.at[pl.ds(qd * _NC, _NC)], startsv)
        iota = lax.iota(jnp.int32, 16)
        zero16 = jnp.zeros((16,), jnp.float32)
        pmax_v = jnp.full((16,), 0, jnp.int32) + pmax

        def task_body(i, _):
            g = g0 + i * 8 + (wid >> 2)
            copies = []
            for j in range(_NT_STAGE):
                csrc = pl.multiple_of(
                    jnp.minimum(cstart + j * 128, _P - 1 - 128), 128)
                copies.append(pltpu.async_copy(
                    x_hbm.at[g, :, pl.ds(csrc, 128)], xq.at[j], sem))
            for j in range(3):
                copies.append(pltpu.async_copy(
                    x_hbm.at[g, :, pl.ds(j * 128, 128)], sb.at[j], sem))
            for c in copies:
                c.wait()

            def frame_body(f, _):
                for q in range(_NCOPY * _NC // 16):
                    accf[pl.ds(q * 16, 16)] = zero16
                fv = jnp.full((16,), 0, jnp.int32) + f

                def chunk_body(kk, carry):
                    kloc = kk * 32
                    res = carry
                    for h in range(2):
                        lo = cidxv[pl.ds(kloc + h * 16, 16)]
                        posv = base0 + kloc + h * 16 + iota
                        m = posv < pmax_v
                        jv = lax.shift_right_logical(posv, 7)
                        lv = posv & 127
                        gat = plsc.load_gather(xq, [jv, fv, lv], mask=m)
                        e = jnp.where(m, jnp.exp(gat), 0.0)
                        plsc.addupdate_scatter(accf, [lo], e)
                        pbuf[pl.ds(kloc + h * 16, 16)] = plsc.cumsum(e) + res
                        res = res + jnp.sum(e)
                    return res

                lax.fori_loop(0, niter, chunk_body, jnp.float32(0.0),
                              unroll=2)

                for q in range(C // 16):
                    ei = endsv[pl.ds(q * 16, 16)]
                    si = startsv[pl.ds(q * 16, 16)]
                    me = ei >= 0
                    ms = si >= 0
                    ge = jnp.where(me, plsc.load_gather(pbuf, [ei], mask=me),
                                   0.0)
                    gs = jnp.where(ms, plsc.load_gather(pbuf, [si], mask=ms),
                                   0.0)
                    col = accf[pl.ds(q * 16, 16)]
                    for cp in range(1, _NCOPY):
                        col = col + accf[pl.ds(cp * _NC + q * 16, 16)]
                    jj, l0 = divmod(q * 16, 128)
                    orow[jj, f, pl.ds(l0, 16)] = (ge - gs) + col

                @pl.when(qd == 0)
                def _():
                    for q in range(C // 16):
                        spos = 1 + q * 16 + iota
                        sjv = lax.shift_right_logical(spos, 7)
                        slv = spos & 127
                        se = jnp.exp(plsc.load_gather(sb, [sjv, fv, slv]))
                        jj, l0 = divmod(q * 16, 128)
                        orow[jj, f, pl.ds(l0, 16)] += se
                return 0

            lax.fori_loop(0, 8, frame_body, 0)
            go = g - g0
            pltpu.sync_copy(orow.at[0],
                            out_hbm.at[qd, pl.ds(go * 8, 8), pl.ds(0, 128)])
            pltpu.sync_copy(orow.at[1],
                            out_hbm.at[qd, pl.ds(go * 8, 8), pl.ds(128, 128)])
            return 0

        lax.fori_loop(0, per, task_body, 0)

    return k(x3, cidx_in, ends_in, starts_in)


_T_SC = 128


def kernel(powerset, mapping):
    B, T, P = powerset.shape
    _, C = mapping.shape
    assert P == _P and C == _NC
    x2 = powerset.reshape(T, P)
    t_tc = T - _T_SC
    x3 = powerset.reshape(T // 8, 8, P)
    parts = _sc_kernel(x3, _T_SC, C, g0=t_tc // 8)
    sc_out = parts[0] + parts[1] + parts[2] + parts[3]
    mnp = _multihot_rows(C, 2)
    mt = jnp.asarray(mnp[P - 1:])
    sc_out = sc_out + jnp.exp(x2[t_tc:, P - 1:]) * mt
    tc_out = _tc_kernel(x2, t_tc, P, C)
    out = jnp.concatenate((tc_out, sc_out), axis=0)
    return out.reshape(B, T, C)
